# baseline (speedup 1.0000x reference)
import functools

import jax
import jax.numpy as jnp
from jax import lax
from jax.experimental import pallas as pl
from jax.experimental.pallas import tpu as pltpu
from jax.experimental.pallas import tpu_sc as plsc

NC = 2
NS = 16
NW = NC * NS
SUB = 128
NPHASE = 1


def _silu(x):
    return x * jax.nn.sigmoid(x)


def _ln(y, g, b, eps=1e-5):
    mu = jnp.mean(y, axis=-1, keepdims=True)
    yc = y - mu
    var = jnp.mean(yc * yc, axis=-1, keepdims=True)
    return yc * lax.rsqrt(var + eps) * g + b


def _worker_span(wid, nrows):
    q, r = nrows // NW, nrows % NW
    ncw = q + jnp.where(wid < r, 1, 0)
    base = wid * q + jnp.minimum(wid, r)
    ab = pl.multiple_of((base // 8) * 8, 8)
    off = base - ab
    return ncw, base, ab, off


def _nload(nrows):
    return (nrows // NW + 8 + 7) // 8 * 8


def _proj_body(xs, xd, wms, wmd, wua, bu1, ps, pd, q):
    ps[...] = jnp.dot(xs[...], wms[...], preferred_element_type=jnp.float32)
    pd[...] = jnp.dot(xd[...], wmd[...], preferred_element_type=jnp.float32)
    q[...] = jnp.dot(xd[...], wua[...], preferred_element_type=jnp.float32) + bu1[...]


def _proj_call(x_src, x_dst, wms, wmd, wua, bu1):
    n, d = x_src.shape
    h = wms.shape[1]
    bn = 2000
    grid = (n // bn,)
    row = pl.BlockSpec((bn, d), lambda i: (i, 0))
    full = lambda a: pl.BlockSpec(a.shape, lambda i: (0, 0))
    return pl.pallas_call(
        _proj_body,
        grid=grid,
        in_specs=[row, row, full(wms), full(wmd), full(wua), full(bu1)],
        out_specs=[pl.BlockSpec((bn, h), lambda i: (i, 0))] * 3,
        out_shape=[jax.ShapeDtypeStruct((n, h), jnp.float32)] * 3,
    )(x_src, x_dst, wms, wmd, wua, bu1)


def _gather_body(psrc, pdst, srcix, dstix, h0, isrc_v, idst_v, a0, b0, a1,
                 b1, sa0, sb0, sa1, sb1, so0, so1):
    c = lax.axis_index("c")
    s = lax.axis_index("s")
    wid = s * NC + c
    h = a0.shape[1]
    nrows = h0.shape[0] // SUB
    ncw, base, ab, off = _worker_span(wid, nrows)
    nload = isrc_v.shape[0]
    pltpu.sync_copy(srcix.at[pl.ds(ab, nload)], isrc_v)
    pltpu.sync_copy(dstix.at[pl.ds(ab, nload)], idst_v)

    def fire(j, av, bv, sa, sb):
        pltpu.async_copy(psrc.at[isrc_v.at[off + j]], av, sa)
        pltpu.async_copy(pdst.at[idst_v.at[off + j]], bv, sb)

    def wait(j, av, bv, sa, sb):
        pltpu.make_async_copy(psrc.at[isrc_v.at[off + j]], av, sa).wait()
        pltpu.make_async_copy(pdst.at[idst_v.at[off + j]], bv, sb).wait()

    def store(j, av, bv, so):
        @plsc.parallel_loop(0, SUB, 1, unroll=2)
        def row(rr):
            for k in range(h // 16):
                sl = pl.ds(k * 16, 16)
                av[rr, sl] = av[rr, sl] + bv[rr, sl]

        row0 = pl.multiple_of((base + j) * SUB, SUB)
        pltpu.async_copy(av, h0.at[pl.ds(row0, SUB)], so)

    def store_wait(av, so):
        row0 = pl.multiple_of(base * SUB, SUB)
        pltpu.make_async_copy(av, h0.at[pl.ds(row0, SUB)], so).wait()

    fire(0, a0, b0, sa0, sb0)

    def body(jj, carry):
        j0 = 2 * jj
        j1 = j0 + 1

        @pl.when(jj > 0)
        def _():
            store_wait(a1, so1)

        fire(j1, a1, b1, sa1, sb1)
        wait(j0, a0, b0, sa0, sb0)
        store(j0, a0, b0, so0)

        @pl.when(j1 + 1 < ncw)
        def _():
            store_wait(a0, so0)
            fire(j1 + 1, a0, b0, sa0, sb0)

        wait(j1, a1, b1, sa1, sb1)
        store(j1, a1, b1, so1)
        return carry

    lax.fori_loop(0, ncw // 2, body, 0)

    @pl.when(ncw % 2 == 1)
    def _():
        j = ncw - 1
        wait(j, a0, b0, sa0, sb0)
        store(j, a0, b0, so0)

    store_wait(a0, so0)

    @pl.when(ncw >= 2)
    def _():
        store_wait(a1, so1)


def _gather_call(psrc, pdst, srcix, dstix, e):
    h = psrc.shape[1]
    nload = _nload(e // SUB)
    mesh = plsc.VectorSubcoreMesh(
        core_axis_name="c", subcore_axis_name="s", num_cores=NC, num_subcores=NS)
    kfn = functools.partial(
        pl.kernel,
        mesh=mesh,
        out_type=jax.ShapeDtypeStruct((e, h), jnp.float32),
        scratch_types=[
            pltpu.VMEM((nload, SUB), jnp.int32),
            pltpu.VMEM((nload, SUB), jnp.int32),
            pltpu.VMEM((SUB, h), jnp.float32),
            pltpu.VMEM((SUB, h), jnp.float32),
            pltpu.VMEM((SUB, h), jnp.float32),
            pltpu.VMEM((SUB, h), jnp.float32),
            pltpu.SemaphoreType.DMA,
            pltpu.SemaphoreType.DMA,
            pltpu.SemaphoreType.DMA,
            pltpu.SemaphoreType.DMA,
            pltpu.SemaphoreType.DMA,
            pltpu.SemaphoreType.DMA,
        ],
    )(_gather_body)
    return kfn(psrc, pdst, srcix, dstix)


def _edge_body(ea, h0, we1, be1, we2, be2, lge, lbe, wme, bm1, wm2, bm2, lgm,
               lbm, out):
    bf = jnp.bfloat16
    y = _silu(jnp.dot(ea[...], we1[...], preferred_element_type=jnp.float32) + be1[...])
    y = jnp.dot(y.astype(bf), we2[...].astype(bf),
                preferred_element_type=jnp.float32) + be2[...]
    e = _ln(y, lge[...], lbe[...])
    pre = (h0[...] +
           jnp.dot(e.astype(bf), wme[...].astype(bf),
                   preferred_element_type=jnp.float32) + bm1[...])
    s = _silu(pre)
    mm = jnp.dot(s.astype(bf), wm2[...].astype(bf),
                 preferred_element_type=jnp.float32) + bm2[...]
    out[...] = _ln(mm, lgm[...], lbm[...])


def _edge_call(ea, h0, we1, be1, we2, be2, lge, lbe, wme, bm1, wm2, bm2, lgm, lbm):
    ep, de = ea.shape
    h = h0.shape[1]
    te = 4000
    grid = (ep // te,)
    full = lambda a: pl.BlockSpec(a.shape, lambda i: (0, 0))
    return pl.pallas_call(
        _edge_body,
        grid=grid,
        in_specs=[
            pl.BlockSpec((te, de), lambda i: (i, 0)),
            pl.BlockSpec((te, h), lambda i: (i, 0)),
            full(we1), full(be1), full(we2), full(be2), full(lge), full(lbe),
            full(wme), full(bm1), full(wm2), full(bm2), full(lgm), full(lbm),
        ],
        out_specs=pl.BlockSpec((te, h), lambda i: (i, 0)),
        out_shape=jax.ShapeDtypeStruct((ep, h), jnp.float32),
    )(ea, h0, we1, be1, we2, be2, lge, lbe, wme, bm1, wm2, bm2, lgm, lbm)


def _scatter_body(m, dstix, zrows, out, idx_v, m0, m1, agg_sh, sm0, sm1):
    c = lax.axis_index("c")
    s = lax.axis_index("s")
    wid = s * NC + c
    nrows = m.shape[0] // SUB
    ncw, base, ab, off = _worker_span(wid, nrows)
    nload = idx_v.shape[0]
    nagg = agg_sh.shape[0]

    zsl = nagg // NS
    pltpu.sync_copy(zrows.at[pl.ds(s * zsl, zsl)], agg_sh.at[pl.ds(s * zsl, zsl)])
    pltpu.sync_copy(dstix.at[pl.ds(ab, nload)], idx_v)
    plsc.subcore_barrier()

    def fire(j, mv, sm):
        row0 = pl.multiple_of((base + j) * SUB, SUB)
        pltpu.async_copy(m.at[pl.ds(row0, SUB)], mv, sm)

    def wait(j, mv, sm):
        row0 = pl.multiple_of((base + j) * SUB, SUB)
        pltpu.make_async_copy(m.at[pl.ds(row0, SUB)], mv, sm).wait()

    def scat(j, mv):
        pltpu.sync_copy(mv, agg_sh.at[idx_v.at[off + j]], add=True)

    fire(0, m0, sm0)

    def body(jj, carry):
        j0 = 2 * jj
        j1 = j0 + 1
        fire(j1, m1, sm1)
        wait(j0, m0, sm0)
        scat(j0, m0)

        @pl.when(j1 + 1 < ncw)
        def _():
            fire(j1 + 1, m0, sm0)

        wait(j1, m1, sm1)
        scat(j1, m1)
        return carry

    lax.fori_loop(0, ncw // 2, body, 0)

    @pl.when(ncw % 2 == 1)
    def _():
        j = ncw - 1
        wait(j, m0, sm0)
        scat(j, m0)

    plsc.subcore_barrier()
    pltpu.sync_copy(agg_sh.at[pl.ds(s * zsl, zsl)], out.at[c, pl.ds(s * zsl, zsl)])


def _scatter_call(m, dstix, nagg):
    e, h = m.shape
    nload = _nload(e // SUB)
    zrows = jnp.zeros((nagg, h), jnp.float32)
    mesh = plsc.VectorSubcoreMesh(
        core_axis_name="c", subcore_axis_name="s", num_cores=NC, num_subcores=NS)
    kfn = functools.partial(
        pl.kernel,
        mesh=mesh,
        out_type=jax.ShapeDtypeStruct((NC, nagg, h), jnp.float32),
        scratch_types=[
            pltpu.VMEM((nload, SUB), jnp.int32),
            pltpu.VMEM((SUB, h), jnp.float32),
            pltpu.VMEM((SUB, h), jnp.float32),
            pltpu.VMEM_SHARED((nagg, h), jnp.float32),
            pltpu.SemaphoreType.DMA,
            pltpu.SemaphoreType.DMA,
        ],
    )(_scatter_body)
    return kfn(m, dstix, zrows)


def _node_body(*refs):
    xd, q = refs[0], refs[1]
    a_refs = refs[2:-4]
    wub, wu2, bu2, out = refs[-4:]
    a = a_refs[0][...]
    for ar in a_refs[1:]:
        a = a + ar[...]
    u = _silu(q[...] + jnp.dot(a, wub[...], preferred_element_type=jnp.float32))
    out[...] = xd[...] + jnp.dot(u, wu2[...], preferred_element_type=jnp.float32) + bu2[...]


def _node_call(x_dst, q, aggs, wub, wu2, bu2):
    n, d = x_dst.shape
    h = wub.shape[0]
    bn = 2000
    grid = (n // bn,)
    row = lambda w: pl.BlockSpec((bn, w), lambda i: (i, 0))
    full = lambda a: pl.BlockSpec(a.shape, lambda i: (0, 0))
    return pl.pallas_call(
        _node_body,
        grid=grid,
        in_specs=[row(d), row(h)] + [row(h)] * len(aggs) +
                 [full(wub), full(wu2), full(bu2)],
        out_specs=row(d),
        out_shape=jax.ShapeDtypeStruct((n, d), jnp.float32),
    )(x_dst, q, *aggs, wub, wu2, bu2)


def kernel(x_src, x_dst, batch_size, edge_index, edge_attr_base, trainable,
           W_e1, b_e1, W_e2, b_e2, ln_ge, ln_be,
           W_m1, b_m1, W_m2, b_m2, ln_gm, ln_bm,
           W_u1, b_u1, W_u2, b_u2):
    n_src, d_src = x_src.shape
    n_dst, d_dst = x_dst.shape
    e = edge_index.shape[1]
    assert e % SUB == 0

    edge_inc = jnp.array([[n_src], [n_dst]], dtype=edge_index.dtype)
    ei = edge_index + (batch_size - 1) * edge_inc
    src, dst = ei[0], ei[1]

    ea = jnp.concatenate([edge_attr_base, trainable], axis=1)

    wms = W_m1[:d_src]
    wmd = W_m1[d_src:d_src + d_dst]
    wme = W_m1[d_src + d_dst:]
    wua = W_u1[:d_dst]
    wub = W_u1[d_dst:]
    r2 = lambda v: v.reshape(1, -1)

    psrc, pdst, q = _proj_call(x_src, x_dst, wms, wmd, wua, r2(b_u1))
    nagg = ((n_dst + NS * 8 - 1) // (NS * 8)) * (NS * 8)

    nphase = NPHASE
    eh = e // nphase
    assert eh % SUB == 0
    phases = []
    for p in range(nphase):
        sl = slice(p * eh, (p + 1) * eh)
        nrows = eh // SUB
        nload = _nload(nrows)
        prow = ((NW - 1) * (nrows // NW) + nrows % NW) // 8 * 8 + nload
        prow = max((prow + 7) // 8 * 8, nrows)
        padr = prow * SUB - eh
        six = jnp.concatenate([src[sl], jnp.zeros((padr,), src.dtype)]).reshape(prow, SUB)
        dix = jnp.concatenate([dst[sl], jnp.zeros((padr,), dst.dtype)]).reshape(prow, SUB)
        phases.append((six, dix, ea[sl]))

    edge = lambda p, h0: _edge_call(
        phases[p][2], h0, W_e1, r2(b_e1), W_e2, r2(b_e2), r2(ln_ge),
        r2(ln_be), wme, r2(b_m1), W_m2, r2(b_m2), r2(ln_gm), r2(ln_bm))

    h0s = [None] * nphase
    ms = [None] * nphase
    aggp = [None] * nphase
    h0s[0] = _gather_call(psrc, pdst, phases[0][0], phases[0][1], eh)
    for p in range(nphase):
        ms[p] = edge(p, h0s[p])
        if p + 1 < nphase:
            h0s[p + 1] = _gather_call(psrc, pdst, phases[p + 1][0],
                                      phases[p + 1][1], eh)
        aggp[p] = _scatter_call(ms[p], phases[p][1], nagg)
    aggs = [a[i, :n_dst] for a in aggp for i in range(NC)]
    x_dst_out = _node_call(x_dst, q, aggs, wub, W_u2, r2(b_u2))
    return (x_src, x_dst_out)


# speedup vs baseline: 1.0536x; 1.0536x over previous
import functools

import jax
import jax.numpy as jnp
from jax import lax
from jax.experimental import pallas as pl
from jax.experimental.pallas import tpu as pltpu
from jax.experimental.pallas import tpu_sc as plsc

NC = 2
NS = 16
NW = NC * NS
SUB = 128
NPHASE = 2


def _silu(x):
    return x * jax.nn.sigmoid(x)


def _ln(y, g, b, eps=1e-5):
    mu = jnp.mean(y, axis=-1, keepdims=True)
    yc = y - mu
    var = jnp.mean(yc * yc, axis=-1, keepdims=True)
    return yc * lax.rsqrt(var + eps) * g + b


def _worker_span(wid, nrows):
    q, r = nrows // NW, nrows % NW
    ncw = q + jnp.where(wid < r, 1, 0)
    base = wid * q + jnp.minimum(wid, r)
    ab = pl.multiple_of((base // 8) * 8, 8)
    off = base - ab
    return ncw, base, ab, off


def _nload(nrows):
    return (nrows // NW + 8 + 7) // 8 * 8


def _proj_body(xs, xd, wms, wmd, ps, pd):
    ps[...] = jnp.dot(xs[...], wms[...], preferred_element_type=jnp.float32)
    pd[...] = jnp.dot(xd[...], wmd[...], preferred_element_type=jnp.float32)


def _proj_call(x_src, x_dst, wms, wmd):
    n, d = x_src.shape
    h = wms.shape[1]
    bn = 2000
    grid = (n // bn,)
    row = pl.BlockSpec((bn, d), lambda i: (i, 0))
    full = lambda a: pl.BlockSpec(a.shape, lambda i: (0, 0))
    return pl.pallas_call(
        _proj_body,
        grid=grid,
        in_specs=[row, row, full(wms), full(wmd)],
        out_specs=[pl.BlockSpec((bn, h), lambda i: (i, 0))] * 2,
        out_shape=[jax.ShapeDtypeStruct((n, h), jnp.float32)] * 2,
    )(x_src, x_dst, wms, wmd)


def _gather_body(psrc, pdst, srcix, dstix, h0, isrc_v, idst_v, a0, b0, a1,
                 b1, sa0, sb0, sa1, sb1, so0, so1):
    c = lax.axis_index("c")
    s = lax.axis_index("s")
    wid = s * NC + c
    h = a0.shape[1]
    nrows = h0.shape[0] // SUB
    ncw, base, ab, off = _worker_span(wid, nrows)
    nload = isrc_v.shape[0]
    pltpu.sync_copy(srcix.at[pl.ds(ab, nload)], isrc_v)
    pltpu.sync_copy(dstix.at[pl.ds(ab, nload)], idst_v)

    def fire(j, av, bv, sa, sb):
        pltpu.async_copy(psrc.at[isrc_v.at[off + j]], av, sa)
        pltpu.async_copy(pdst.at[idst_v.at[off + j]], bv, sb)

    def wait(j, av, bv, sa, sb):
        pltpu.make_async_copy(psrc.at[isrc_v.at[off + j]], av, sa).wait()
        pltpu.make_async_copy(pdst.at[idst_v.at[off + j]], bv, sb).wait()

    def store(j, av, bv, so):
        @plsc.parallel_loop(0, SUB, 1, unroll=2)
        def row(rr):
            for k in range(h // 16):
                sl = pl.ds(k * 16, 16)
                av[rr, sl] = av[rr, sl] + bv[rr, sl]

        row0 = pl.multiple_of((base + j) * SUB, SUB)
        pltpu.async_copy(av, h0.at[pl.ds(row0, SUB)], so)

    def store_wait(av, so):
        row0 = pl.multiple_of(base * SUB, SUB)
        pltpu.make_async_copy(av, h0.at[pl.ds(row0, SUB)], so).wait()

    fire(0, a0, b0, sa0, sb0)

    def body(jj, carry):
        j0 = 2 * jj
        j1 = j0 + 1

        @pl.when(jj > 0)
        def _():
            store_wait(a1, so1)

        fire(j1, a1, b1, sa1, sb1)
        wait(j0, a0, b0, sa0, sb0)
        store(j0, a0, b0, so0)

        @pl.when(j1 + 1 < ncw)
        def _():
            store_wait(a0, so0)
            fire(j1 + 1, a0, b0, sa0, sb0)

        wait(j1, a1, b1, sa1, sb1)
        store(j1, a1, b1, so1)
        return carry

    lax.fori_loop(0, ncw // 2, body, 0)

    @pl.when(ncw % 2 == 1)
    def _():
        j = ncw - 1
        wait(j, a0, b0, sa0, sb0)
        store(j, a0, b0, so0)

    store_wait(a0, so0)

    @pl.when(ncw >= 2)
    def _():
        store_wait(a1, so1)


def _gather_call(psrc, pdst, srcix, dstix, e):
    h = psrc.shape[1]
    nload = _nload(e // SUB)
    mesh = plsc.VectorSubcoreMesh(
        core_axis_name="c", subcore_axis_name="s", num_cores=NC, num_subcores=NS)
    kfn = functools.partial(
        pl.kernel,
        mesh=mesh,
        out_type=jax.ShapeDtypeStruct((e, h), jnp.float32),
        scratch_types=[
            pltpu.VMEM((nload, SUB), jnp.int32),
            pltpu.VMEM((nload, SUB), jnp.int32),
            pltpu.VMEM((SUB, h), jnp.float32),
            pltpu.VMEM((SUB, h), jnp.float32),
            pltpu.VMEM((SUB, h), jnp.float32),
            pltpu.VMEM((SUB, h), jnp.float32),
            pltpu.SemaphoreType.DMA,
            pltpu.SemaphoreType.DMA,
            pltpu.SemaphoreType.DMA,
            pltpu.SemaphoreType.DMA,
            pltpu.SemaphoreType.DMA,
            pltpu.SemaphoreType.DMA,
        ],
    )(_gather_body)
    return kfn(psrc, pdst, srcix, dstix)


def _edge_body(ea, h0, we1, be1, we2, be2, lge, lbe, wme, bm1, wm2, bm2, lgm,
               lbm, out):
    bf = jnp.bfloat16
    y = _silu(jnp.dot(ea[...], we1[...], preferred_element_type=jnp.float32) + be1[...])
    y = jnp.dot(y.astype(bf), we2[...].astype(bf),
                preferred_element_type=jnp.float32) + be2[...]
    e = _ln(y, lge[...], lbe[...])
    pre = (h0[...] +
           jnp.dot(e.astype(bf), wme[...].astype(bf),
                   preferred_element_type=jnp.float32) + bm1[...])
    s = _silu(pre)
    mm = jnp.dot(s.astype(bf), wm2[...].astype(bf),
                 preferred_element_type=jnp.float32) + bm2[...]
    out[...] = _ln(mm, lgm[...], lbm[...])


def _edge_call(ea, h0, we1, be1, we2, be2, lge, lbe, wme, bm1, wm2, bm2, lgm, lbm):
    ep, de = ea.shape
    h = h0.shape[1]
    te = 4000
    grid = (ep // te,)
    full = lambda a: pl.BlockSpec(a.shape, lambda i: (0, 0))
    return pl.pallas_call(
        _edge_body,
        grid=grid,
        in_specs=[
            pl.BlockSpec((te, de), lambda i: (i, 0)),
            pl.BlockSpec((te, h), lambda i: (i, 0)),
            full(we1), full(be1), full(we2), full(be2), full(lge), full(lbe),
            full(wme), full(bm1), full(wm2), full(bm2), full(lgm), full(lbm),
        ],
        out_specs=pl.BlockSpec((te, h), lambda i: (i, 0)),
        out_shape=jax.ShapeDtypeStruct((ep, h), jnp.float32),
    )(ea, h0, we1, be1, we2, be2, lge, lbe, wme, bm1, wm2, bm2, lgm, lbm)


def _scatter_body(m, dstix, zrows, out, idx_v, m0, m1, agg_sh, sm0, sm1):
    c = lax.axis_index("c")
    s = lax.axis_index("s")
    wid = s * NC + c
    nrows = m.shape[0] // SUB
    ncw, base, ab, off = _worker_span(wid, nrows)
    nload = idx_v.shape[0]
    nagg = agg_sh.shape[0]

    zsl = nagg // NS
    pltpu.sync_copy(zrows.at[pl.ds(s * zsl, zsl)], agg_sh.at[pl.ds(s * zsl, zsl)])
    pltpu.sync_copy(dstix.at[pl.ds(ab, nload)], idx_v)
    plsc.subcore_barrier()

    def fire(j, mv, sm):
        row0 = pl.multiple_of((base + j) * SUB, SUB)
        pltpu.async_copy(m.at[pl.ds(row0, SUB)], mv, sm)

    def wait(j, mv, sm):
        row0 = pl.multiple_of((base + j) * SUB, SUB)
        pltpu.make_async_copy(m.at[pl.ds(row0, SUB)], mv, sm).wait()

    def scat(j, mv):
        pltpu.sync_copy(mv, agg_sh.at[idx_v.at[off + j]], add=True)

    fire(0, m0, sm0)

    def body(jj, carry):
        j0 = 2 * jj
        j1 = j0 + 1
        fire(j1, m1, sm1)
        wait(j0, m0, sm0)
        scat(j0, m0)

        @pl.when(j1 + 1 < ncw)
        def _():
            fire(j1 + 1, m0, sm0)

        wait(j1, m1, sm1)
        scat(j1, m1)
        return carry

    lax.fori_loop(0, ncw // 2, body, 0)

    @pl.when(ncw % 2 == 1)
    def _():
        j = ncw - 1
        wait(j, m0, sm0)
        scat(j, m0)

    plsc.subcore_barrier()
    pltpu.sync_copy(agg_sh.at[pl.ds(s * zsl, zsl)], out.at[c, pl.ds(s * zsl, zsl)])


def _scatter_call(m, dstix, nagg):
    e, h = m.shape
    nload = _nload(e // SUB)
    zrows = jnp.zeros((nagg, h), jnp.float32)
    mesh = plsc.VectorSubcoreMesh(
        core_axis_name="c", subcore_axis_name="s", num_cores=NC, num_subcores=NS)
    kfn = functools.partial(
        pl.kernel,
        mesh=mesh,
        out_type=jax.ShapeDtypeStruct((NC, nagg, h), jnp.float32),
        scratch_types=[
            pltpu.VMEM((nload, SUB), jnp.int32),
            pltpu.VMEM((SUB, h), jnp.float32),
            pltpu.VMEM((SUB, h), jnp.float32),
            pltpu.VMEM_SHARED((nagg, h), jnp.float32),
            pltpu.SemaphoreType.DMA,
            pltpu.SemaphoreType.DMA,
        ],
    )(_scatter_body)
    return kfn(m, dstix, zrows)


def _node_body(*refs):
    xd = refs[0]
    a_refs = refs[1:-6]
    wua, bu1, wub, wu2, bu2, out = refs[-6:]
    a = a_refs[0][...]
    for ar in a_refs[1:]:
        a = a + ar[...]
    u = _silu(jnp.dot(xd[...], wua[...], preferred_element_type=jnp.float32) +
              bu1[...] +
              jnp.dot(a, wub[...], preferred_element_type=jnp.float32))
    out[...] = xd[...] + jnp.dot(u, wu2[...], preferred_element_type=jnp.float32) + bu2[...]


def _node_call(x_dst, aggs, wua, bu1, wub, wu2, bu2):
    n, d = x_dst.shape
    h = wub.shape[0]
    bn = 2000
    grid = (n // bn,)
    row = lambda w: pl.BlockSpec((bn, w), lambda i: (i, 0))
    full = lambda a: pl.BlockSpec(a.shape, lambda i: (0, 0))
    return pl.pallas_call(
        _node_body,
        grid=grid,
        in_specs=[row(d)] + [row(h)] * len(aggs) +
                 [full(wua), full(bu1), full(wub), full(wu2), full(bu2)],
        out_specs=row(d),
        out_shape=jax.ShapeDtypeStruct((n, d), jnp.float32),
    )(x_dst, *aggs, wua, bu1, wub, wu2, bu2)


def kernel(x_src, x_dst, batch_size, edge_index, edge_attr_base, trainable,
           W_e1, b_e1, W_e2, b_e2, ln_ge, ln_be,
           W_m1, b_m1, W_m2, b_m2, ln_gm, ln_bm,
           W_u1, b_u1, W_u2, b_u2):
    n_src, d_src = x_src.shape
    n_dst, d_dst = x_dst.shape
    e = edge_index.shape[1]
    assert e % SUB == 0

    edge_inc = jnp.array([[n_src], [n_dst]], dtype=edge_index.dtype)
    ei = edge_index + (batch_size - 1) * edge_inc
    src, dst = ei[0], ei[1]

    ea = jnp.concatenate([edge_attr_base, trainable], axis=1)

    wms = W_m1[:d_src]
    wmd = W_m1[d_src:d_src + d_dst]
    wme = W_m1[d_src + d_dst:]
    wua = W_u1[:d_dst]
    wub = W_u1[d_dst:]
    r2 = lambda v: v.reshape(1, -1)

    psrc, pdst = _proj_call(x_src, x_dst, wms, wmd)
    nagg = ((n_dst + NS * 8 - 1) // (NS * 8)) * (NS * 8)

    nphase = NPHASE
    eh = e // nphase
    assert eh % SUB == 0
    phases = []
    for p in range(nphase):
        sl = slice(p * eh, (p + 1) * eh)
        nrows = eh // SUB
        nload = _nload(nrows)
        prow = ((NW - 1) * (nrows // NW) + nrows % NW) // 8 * 8 + nload
        prow = max((prow + 7) // 8 * 8, nrows)
        padr = prow * SUB - eh
        six = jnp.concatenate([src[sl], jnp.zeros((padr,), src.dtype)]).reshape(prow, SUB)
        dix = jnp.concatenate([dst[sl], jnp.zeros((padr,), dst.dtype)]).reshape(prow, SUB)
        phases.append((six, dix, ea[sl]))

    edge = lambda p, h0: _edge_call(
        phases[p][2], h0, W_e1, r2(b_e1), W_e2, r2(b_e2), r2(ln_ge),
        r2(ln_be), wme, r2(b_m1), W_m2, r2(b_m2), r2(ln_gm), r2(ln_bm))

    h0s = [None] * nphase
    ms = [None] * nphase
    aggp = [None] * nphase
    h0s[0] = _gather_call(psrc, pdst, phases[0][0], phases[0][1], eh)
    for p in range(nphase):
        ms[p] = edge(p, h0s[p])
        if p + 1 < nphase:
            h0s[p + 1] = _gather_call(psrc, pdst, phases[p + 1][0],
                                      phases[p + 1][1], eh)
        aggp[p] = _scatter_call(ms[p], phases[p][1], nagg)
    aggs = [a[i, :n_dst] for a in aggp for i in range(NC)]
    x_dst_out = _node_call(x_dst, aggs, wua, r2(b_u1), wub, W_u2, r2(b_u2))
    return (x_src, x_dst_out)


# speedup vs baseline: 1.0707x; 1.0162x over previous
import functools

import jax
import jax.numpy as jnp
from jax import lax
from jax.experimental import pallas as pl
from jax.experimental.pallas import tpu as pltpu
from jax.experimental.pallas import tpu_sc as plsc

NC = 2
NS = 16
NW = NC * NS
SUB = 128
NPHASE = 2


def _silu(x):
    return x * jax.nn.sigmoid(x)


def _ln(y, g, b, eps=1e-5):
    mu = jnp.mean(y, axis=-1, keepdims=True)
    yc = y - mu
    var = jnp.mean(yc * yc, axis=-1, keepdims=True)
    return yc * lax.rsqrt(var + eps) * g + b


def _worker_span(wid, nrows):
    q, r = nrows // NW, nrows % NW
    ncw = q + jnp.where(wid < r, 1, 0)
    base = wid * q + jnp.minimum(wid, r)
    ab = pl.multiple_of((base // 8) * 8, 8)
    off = base - ab
    return ncw, base, ab, off


def _nload(nrows):
    return (nrows // NW + 8 + 7) // 8 * 8


def _proj_body(xs, xd, wms, wmd, ps, pd):
    ps[...] = jnp.dot(xs[...], wms[...], preferred_element_type=jnp.float32)
    pd[...] = jnp.dot(xd[...], wmd[...], preferred_element_type=jnp.float32)


def _proj_call(x_src, x_dst, wms, wmd):
    n, d = x_src.shape
    h = wms.shape[1]
    bn = 2000
    grid = (n // bn,)
    row = pl.BlockSpec((bn, d), lambda i: (i, 0))
    full = lambda a: pl.BlockSpec(a.shape, lambda i: (0, 0))
    return pl.pallas_call(
        _proj_body,
        grid=grid,
        in_specs=[row, row, full(wms), full(wmd)],
        out_specs=[pl.BlockSpec((bn, h), lambda i: (i, 0))] * 2,
        out_shape=[jax.ShapeDtypeStruct((n, h), jnp.float32)] * 2,
    )(x_src, x_dst, wms, wmd)


def _gather_body(psrc, pdst, srcix, dstix, h0, isrc_v, idst_v, a0, b0, a1,
                 b1, sa0, sb0, sa1, sb1, so0, so1):
    c = lax.axis_index("c")
    s = lax.axis_index("s")
    wid = s * NC + c
    h = a0.shape[1]
    nrows = h0.shape[0] // SUB
    ncw, base, ab, off = _worker_span(wid, nrows)
    nload = isrc_v.shape[0]
    pltpu.sync_copy(srcix.at[pl.ds(ab, nload)], isrc_v)
    pltpu.sync_copy(dstix.at[pl.ds(ab, nload)], idst_v)

    def fire(j, av, bv, sa, sb):
        pltpu.async_copy(psrc.at[isrc_v.at[off + j]], av, sa)
        pltpu.async_copy(pdst.at[idst_v.at[off + j]], bv, sb)

    def wait(j, av, bv, sa, sb):
        pltpu.make_async_copy(psrc.at[isrc_v.at[off + j]], av, sa).wait()
        pltpu.make_async_copy(pdst.at[idst_v.at[off + j]], bv, sb).wait()

    def store(j, av, bv, so):
        @plsc.parallel_loop(0, SUB, 1, unroll=2)
        def row(rr):
            for k in range(h // 16):
                sl = pl.ds(k * 16, 16)
                av[rr, sl] = av[rr, sl] + bv[rr, sl]

        row0 = pl.multiple_of((base + j) * SUB, SUB)
        pltpu.async_copy(av, h0.at[pl.ds(row0, SUB)], so)

    def store_wait(av, so):
        row0 = pl.multiple_of(base * SUB, SUB)
        pltpu.make_async_copy(av, h0.at[pl.ds(row0, SUB)], so).wait()

    fire(0, a0, b0, sa0, sb0)

    def body(jj, carry):
        j0 = 2 * jj
        j1 = j0 + 1

        @pl.when(jj > 0)
        def _():
            store_wait(a1, so1)

        fire(j1, a1, b1, sa1, sb1)
        wait(j0, a0, b0, sa0, sb0)
        store(j0, a0, b0, so0)

        @pl.when(j1 + 1 < ncw)
        def _():
            store_wait(a0, so0)
            fire(j1 + 1, a0, b0, sa0, sb0)

        wait(j1, a1, b1, sa1, sb1)
        store(j1, a1, b1, so1)
        return carry

    lax.fori_loop(0, ncw // 2, body, 0)

    @pl.when(ncw % 2 == 1)
    def _():
        j = ncw - 1
        wait(j, a0, b0, sa0, sb0)
        store(j, a0, b0, so0)

    store_wait(a0, so0)

    @pl.when(ncw >= 2)
    def _():
        store_wait(a1, so1)


def _gather_call(psrc, pdst, srcix, dstix, e):
    h = psrc.shape[1]
    nload = _nload(e // SUB)
    mesh = plsc.VectorSubcoreMesh(
        core_axis_name="c", subcore_axis_name="s", num_cores=NC, num_subcores=NS)
    kfn = functools.partial(
        pl.kernel,
        mesh=mesh,
        out_type=jax.ShapeDtypeStruct((e, h), jnp.float32),
        scratch_types=[
            pltpu.VMEM((nload, SUB), jnp.int32),
            pltpu.VMEM((nload, SUB), jnp.int32),
            pltpu.VMEM((SUB, h), jnp.float32),
            pltpu.VMEM((SUB, h), jnp.float32),
            pltpu.VMEM((SUB, h), jnp.float32),
            pltpu.VMEM((SUB, h), jnp.float32),
            pltpu.SemaphoreType.DMA,
            pltpu.SemaphoreType.DMA,
            pltpu.SemaphoreType.DMA,
            pltpu.SemaphoreType.DMA,
            pltpu.SemaphoreType.DMA,
            pltpu.SemaphoreType.DMA,
        ],
    )(_gather_body)
    return kfn(psrc, pdst, srcix, dstix)


def _edge_body(ea, h0, we1, be1, we2, be2, lge, lbe, wme, bm1, wm2, bm2, lgm,
               lbm, out):
    bf = jnp.bfloat16
    y = _silu(jnp.dot(ea[...], we1[...], preferred_element_type=jnp.float32) + be1[...])
    y = jnp.dot(y.astype(bf), we2[...].astype(bf),
                preferred_element_type=jnp.float32) + be2[...]
    e = _ln(y, lge[...], lbe[...])
    pre = (h0[...] +
           jnp.dot(e.astype(bf), wme[...].astype(bf),
                   preferred_element_type=jnp.float32) + bm1[...])
    s = _silu(pre)
    mm = jnp.dot(s.astype(bf), wm2[...].astype(bf),
                 preferred_element_type=jnp.float32) + bm2[...]
    out[...] = _ln(mm, lgm[...], lbm[...])


def _edge_call(ea, h0, we1, be1, we2, be2, lge, lbe, wme, bm1, wm2, bm2, lgm, lbm):
    ep, de = ea.shape
    h = h0.shape[1]
    te = 8000
    grid = (ep // te,)
    full = lambda a: pl.BlockSpec(a.shape, lambda i: (0, 0))
    return pl.pallas_call(
        _edge_body,
        grid=grid,
        in_specs=[
            pl.BlockSpec((te, de), lambda i: (i, 0)),
            pl.BlockSpec((te, h), lambda i: (i, 0)),
            full(we1), full(be1), full(we2), full(be2), full(lge), full(lbe),
            full(wme), full(bm1), full(wm2), full(bm2), full(lgm), full(lbm),
        ],
        out_specs=pl.BlockSpec((te, h), lambda i: (i, 0)),
        out_shape=jax.ShapeDtypeStruct((ep, h), jnp.float32),
    )(ea, h0, we1, be1, we2, be2, lge, lbe, wme, bm1, wm2, bm2, lgm, lbm)


def _scatter_body(m, dstix, zrows, out, idx_v, m0, m1, agg_sh, sm0, sm1):
    c = lax.axis_index("c")
    s = lax.axis_index("s")
    wid = s * NC + c
    nrows = m.shape[0] // SUB
    ncw, base, ab, off = _worker_span(wid, nrows)
    nload = idx_v.shape[0]
    nagg = agg_sh.shape[0]

    zsl = nagg // NS
    pltpu.sync_copy(zrows.at[pl.ds(s * zsl, zsl)], agg_sh.at[pl.ds(s * zsl, zsl)])
    pltpu.sync_copy(dstix.at[pl.ds(ab, nload)], idx_v)
    plsc.subcore_barrier()

    def fire(j, mv, sm):
        row0 = pl.multiple_of((base + j) * SUB, SUB)
        pltpu.async_copy(m.at[pl.ds(row0, SUB)], mv, sm)

    def wait(j, mv, sm):
        row0 = pl.multiple_of((base + j) * SUB, SUB)
        pltpu.make_async_copy(m.at[pl.ds(row0, SUB)], mv, sm).wait()

    def scat(j, mv):
        pltpu.sync_copy(mv, agg_sh.at[idx_v.at[off + j]], add=True)

    fire(0, m0, sm0)

    def body(jj, carry):
        j0 = 2 * jj
        j1 = j0 + 1
        fire(j1, m1, sm1)
        wait(j0, m0, sm0)
        scat(j0, m0)

        @pl.when(j1 + 1 < ncw)
        def _():
            fire(j1 + 1, m0, sm0)

        wait(j1, m1, sm1)
        scat(j1, m1)
        return carry

    lax.fori_loop(0, ncw // 2, body, 0)

    @pl.when(ncw % 2 == 1)
    def _():
        j = ncw - 1
        wait(j, m0, sm0)
        scat(j, m0)

    plsc.subcore_barrier()
    pltpu.sync_copy(agg_sh.at[pl.ds(s * zsl, zsl)], out.at[c, pl.ds(s * zsl, zsl)])


def _scatter_call(m, dstix, nagg):
    e, h = m.shape
    nload = _nload(e // SUB)
    zrows = jnp.zeros((nagg, h), jnp.float32)
    mesh = plsc.VectorSubcoreMesh(
        core_axis_name="c", subcore_axis_name="s", num_cores=NC, num_subcores=NS)
    kfn = functools.partial(
        pl.kernel,
        mesh=mesh,
        out_type=jax.ShapeDtypeStruct((NC, nagg, h), jnp.float32),
        scratch_types=[
            pltpu.VMEM((nload, SUB), jnp.int32),
            pltpu.VMEM((SUB, h), jnp.float32),
            pltpu.VMEM((SUB, h), jnp.float32),
            pltpu.VMEM_SHARED((nagg, h), jnp.float32),
            pltpu.SemaphoreType.DMA,
            pltpu.SemaphoreType.DMA,
        ],
    )(_scatter_body)
    return kfn(m, dstix, zrows)


def _node_body(*refs):
    xd = refs[0]
    a_refs = refs[1:-6]
    wua, bu1, wub, wu2, bu2, out = refs[-6:]
    a = a_refs[0][...]
    for ar in a_refs[1:]:
        a = a + ar[...]
    u = _silu(jnp.dot(xd[...], wua[...], preferred_element_type=jnp.float32) +
              bu1[...] +
              jnp.dot(a, wub[...], preferred_element_type=jnp.float32))
    out[...] = xd[...] + jnp.dot(u, wu2[...], preferred_element_type=jnp.float32) + bu2[...]


def _node_call(x_dst, aggs, wua, bu1, wub, wu2, bu2):
    n, d = x_dst.shape
    h = wub.shape[0]
    bn = 2000
    grid = (n // bn,)
    row = lambda w: pl.BlockSpec((bn, w), lambda i: (i, 0))
    full = lambda a: pl.BlockSpec(a.shape, lambda i: (0, 0))
    return pl.pallas_call(
        _node_body,
        grid=grid,
        in_specs=[row(d)] + [row(h)] * len(aggs) +
                 [full(wua), full(bu1), full(wub), full(wu2), full(bu2)],
        out_specs=row(d),
        out_shape=jax.ShapeDtypeStruct((n, d), jnp.float32),
    )(x_dst, *aggs, wua, bu1, wub, wu2, bu2)


def kernel(x_src, x_dst, batch_size, edge_index, edge_attr_base, trainable,
           W_e1, b_e1, W_e2, b_e2, ln_ge, ln_be,
           W_m1, b_m1, W_m2, b_m2, ln_gm, ln_bm,
           W_u1, b_u1, W_u2, b_u2):
    n_src, d_src = x_src.shape
    n_dst, d_dst = x_dst.shape
    e = edge_index.shape[1]
    assert e % SUB == 0

    edge_inc = jnp.array([[n_src], [n_dst]], dtype=edge_index.dtype)
    ei = edge_index + (batch_size - 1) * edge_inc
    src, dst = ei[0], ei[1]

    ea = jnp.concatenate([edge_attr_base, trainable], axis=1)

    wms = W_m1[:d_src]
    wmd = W_m1[d_src:d_src + d_dst]
    wme = W_m1[d_src + d_dst:]
    wua = W_u1[:d_dst]
    wub = W_u1[d_dst:]
    r2 = lambda v: v.reshape(1, -1)

    psrc, pdst = _proj_call(x_src, x_dst, wms, wmd)
    nagg = ((n_dst + NS * 8 - 1) // (NS * 8)) * (NS * 8)

    nphase = NPHASE
    eh = e // nphase
    assert eh % SUB == 0
    phases = []
    for p in range(nphase):
        sl = slice(p * eh, (p + 1) * eh)
        nrows = eh // SUB
        nload = _nload(nrows)
        prow = ((NW - 1) * (nrows // NW) + nrows % NW) // 8 * 8 + nload
        prow = max((prow + 7) // 8 * 8, nrows)
        padr = prow * SUB - eh
        six = jnp.concatenate([src[sl], jnp.zeros((padr,), src.dtype)]).reshape(prow, SUB)
        dix = jnp.concatenate([dst[sl], jnp.zeros((padr,), dst.dtype)]).reshape(prow, SUB)
        phases.append((six, dix, ea[sl]))

    edge = lambda p, h0: _edge_call(
        phases[p][2], h0, W_e1, r2(b_e1), W_e2, r2(b_e2), r2(ln_ge),
        r2(ln_be), wme, r2(b_m1), W_m2, r2(b_m2), r2(ln_gm), r2(ln_bm))

    h0s = [None] * nphase
    ms = [None] * nphase
    aggp = [None] * nphase
    h0s[0] = _gather_call(psrc, pdst, phases[0][0], phases[0][1], eh)
    for p in range(nphase):
        ms[p] = edge(p, h0s[p])
        if p + 1 < nphase:
            h0s[p + 1] = _gather_call(psrc, pdst, phases[p + 1][0],
                                      phases[p + 1][1], eh)
        aggp[p] = _scatter_call(ms[p], phases[p][1], nagg)
    aggs = [a[i, :n_dst] for a in aggp for i in range(NC)]
    x_dst_out = _node_call(x_dst, aggs, wua, r2(b_u1), wub, W_u2, r2(b_u2))
    return (x_src, x_dst_out)
